# Initial kernel scaffold; baseline (speedup 1.0000x reference)
import jax
import jax.numpy as jnp
from jax.experimental import pallas as pl


def kernel(x, embedding):
    raise NotImplementedError("write your pallas kernel here")


# speedup vs baseline: 1.2595x; 1.2595x over previous
import jax
import jax.numpy as jnp
from jax.experimental import pallas as pl

_COMMITMENT_COST = 0.25
_N_TOK = 16 * 1024
_M = 512
_D = 64
_BLK = 1024
_GRID = _N_TOK // _BLK


def _rowsum64(sq):
    t = sq[:, 0:8]
    for k in range(1, 8):
        t = t + sq[:, 8 * k:8 * k + 8]
    t = t[:, 0:4] + t[:, 4:8]
    t = t[:, 0:2] + t[:, 2:4]
    return t[:, 0:1] + t[:, 1:2]


def _vq_body(x_ref, emb_ref, qst_ref, loss_ref, ppl_ref, counts_ref):
    i = pl.program_id(0)
    x = x_ref[...]
    emb = emb_ref[...]

    x2 = _rowsum64(x * x)
    e2 = _rowsum64(emb * emb).reshape(1, _M)
    dot = jax.lax.dot_general(x, emb, (((1,), (1,)), ((), ())),
                              preferred_element_type=jnp.float32)
    d2 = x2 + e2 - 2.0 * dot
    dist = jnp.maximum(d2, 0.0)
    lanes = jax.lax.broadcasted_iota(jnp.int32, (_BLK, _M), 1)
    minval = jnp.min(dist, axis=1, keepdims=True)
    idx = jnp.min(jnp.where(dist == minval, lanes, _M), axis=1)

    enc = (lanes == idx[:, None]).astype(jnp.float32)
    q = jax.lax.dot_general(enc, emb, (((1,), (0,)), ((), ())),
                            preferred_element_type=jnp.float32)
    qst_ref[...] = x + (q - x)

    diff = x - q
    part_loss = jnp.sum(diff * diff)
    part_counts = jnp.sum(enc, axis=0)[None, :]

    @pl.when(i == 0)
    def _init():
        loss_ref[...] = jnp.zeros_like(loss_ref)
        counts_ref[...] = jnp.zeros_like(counts_ref)
        ppl_ref[...] = jnp.zeros_like(ppl_ref)

    loss_ref[...] += part_loss.reshape(1, 1)
    counts_ref[...] += part_counts

    @pl.when(i == _GRID - 1)
    def _finalize():
        loss_ref[...] = _COMMITMENT_COST * (loss_ref[...] / (_N_TOK * _D))
        p = counts_ref[...] * (1.0 / _N_TOK)
        ppl_ref[...] = jnp.exp(-jnp.sum(p * jnp.log(p + 1e-10))).reshape(1, 1)


def kernel(x, embedding):
    x_flat = x.reshape(_N_TOK, _D)
    qst, loss, ppl, _counts = pl.pallas_call(
        _vq_body,
        grid=(_GRID,),
        in_specs=[
            pl.BlockSpec((_BLK, _D), lambda i: (i, 0)),
            pl.BlockSpec((_M, _D), lambda i: (0, 0)),
        ],
        out_specs=[
            pl.BlockSpec((_BLK, _D), lambda i: (i, 0)),
            pl.BlockSpec((1, 1), lambda i: (0, 0)),
            pl.BlockSpec((1, 1), lambda i: (0, 0)),
            pl.BlockSpec((1, _M), lambda i: (0, 0)),
        ],
        out_shape=[
            jax.ShapeDtypeStruct((_N_TOK, _D), jnp.float32),
            jax.ShapeDtypeStruct((1, 1), jnp.float32),
            jax.ShapeDtypeStruct((1, 1), jnp.float32),
            jax.ShapeDtypeStruct((1, _M), jnp.float32),
        ],
    )(x_flat, embedding)
    return qst.reshape(x.shape), loss[0, 0], ppl[0, 0]
